# Initial kernel scaffold; baseline (speedup 1.0000x reference)
import jax
import jax.numpy as jnp
from jax.experimental import pallas as pl


def kernel(hidden_states, input_ids, attractors, running_mean, running_var):
    raise NotImplementedError("write your pallas kernel here")


# speedup vs baseline: 5.2740x; 5.2740x over previous
import jax
import jax.numpy as jnp
from jax.experimental import pallas as pl

_B, _S, _DIM, _K, _NTYPES, _VOCAB = 4, 4096, 2048, 10, 3, 32000
_ALPHA_BASE, _MAX_DELTA = 0.3, 0.5
_TK = _NTYPES * _K

_BS = 512


def _fused_kernel(x_ref, ids_ref, a_row_ref, a_col_ref, mean_ref, var_ref,
                  o_ref):
    i = pl.program_id(0)
    x = x_ref[...]
    mean = mean_ref[...]
    var = var_ref[...]
    std = jnp.sqrt(var) + 1e-8
    w = (x - mean) / std
    nrm = jnp.sqrt(jnp.sum(w * w, axis=1, keepdims=True))
    h = w / jnp.maximum(nrm, 1e-12)

    a_col = a_col_ref[...]
    cn = jnp.sqrt(jnp.sum(a_col * a_col, axis=0, keepdims=True))
    a_col_n = a_col / jnp.maximum(cn, 1e-12)
    a_row = a_row_ref[...]
    rn = jnp.sqrt(jnp.sum(a_row * a_row, axis=1, keepdims=True))
    a_row_n = a_row / jnp.maximum(rn, 1e-12)

    sims = jnp.dot(h, a_col_n, preferred_element_type=jnp.float32)

    r0 = jax.lax.rem(i, jnp.int32(_S // _BS)) * _BS
    s_idx = r0 + jax.lax.broadcasted_iota(jnp.int32, (_BS, 1), 0)
    rel = s_idx.astype(jnp.float32) / jnp.float32(_S)
    ptype = jnp.where(rel < 0.3, 0, jnp.where(rel < 0.7, 1, 2))
    col = jax.lax.broadcasted_iota(jnp.int32, (1, _TK), 1)
    valid = (col // _K) == ptype
    masked = jnp.where(valid, sims, -jnp.inf)
    mx = jnp.max(masked, axis=1, keepdims=True)
    first = jnp.min(jnp.where(masked == mx, col, _TK), axis=1, keepdims=True)
    onehot = (col == first).astype(jnp.float32)
    nearest = jnp.dot(onehot, a_row_n,
                      preferred_element_type=jnp.float32)

    ids = ids_ref[...]
    is_marker = jax.lax.rem(ids, jnp.int32(500)) == 0
    wgt = jnp.where(is_marker, 5.0, jnp.where(rel > 0.7, 2.0, 1.0))
    alpha = _ALPHA_BASE * wgt.astype(jnp.float32)

    delta = alpha * (nearest - h)
    dn = jnp.sqrt(jnp.sum(delta * delta, axis=1, keepdims=True))
    scale = jnp.minimum(1.0, _MAX_DELTA / jnp.maximum(dn, 1e-12))
    o_ref[...] = x + delta * (scale * std)


@jax.jit
def kernel(hidden_states, input_ids, attractors, running_mean, running_var):
    Bb, Ss, Dd = hidden_states.shape
    x2 = hidden_states.reshape(Bb * Ss, Dd)
    ids2 = input_ids.reshape(Bb * Ss, 1)
    a_row = attractors.reshape(_NTYPES * _K, Dd)
    a_col = a_row.T
    mean2 = running_mean.reshape(1, Dd)
    var2 = running_var.reshape(1, Dd)
    nblk = (Bb * Ss) // _BS
    out = pl.pallas_call(
        _fused_kernel,
        grid=(nblk,),
        in_specs=[
            pl.BlockSpec((_BS, Dd), lambda i: (i, 0)),
            pl.BlockSpec((_BS, 1), lambda i: (i, 0)),
            pl.BlockSpec((_TK, Dd), lambda i: (0, 0)),
            pl.BlockSpec((Dd, _TK), lambda i: (0, 0)),
            pl.BlockSpec((1, Dd), lambda i: (0, 0)),
            pl.BlockSpec((1, Dd), lambda i: (0, 0)),
        ],
        out_specs=pl.BlockSpec((_BS, Dd), lambda i: (i, 0)),
        out_shape=jax.ShapeDtypeStruct((Bb * Ss, Dd), jnp.float32),
    )(x2, ids2, a_row, a_col, mean2, var2)
    return out.reshape(Bb, Ss, Dd)


# speedup vs baseline: 5.8421x; 1.1077x over previous
import jax
import jax.numpy as jnp
from jax.experimental import pallas as pl
from jax.experimental.pallas import tpu as pltpu

_B, _S, _DIM, _K, _NTYPES, _VOCAB = 4, 4096, 2048, 10, 3, 32000
_ALPHA_BASE, _MAX_DELTA = 0.3, 0.5
_TK = _NTYPES * _K

_BS = 512


def _fused_kernel(x_ref, ids_ref, a_row_ref, a_col_ref, mean_ref,
                  var_row_ref, var_col_ref, o_ref,
                  bhat_ref, atil_ref, vtil_ref):
    i = pl.program_id(0)

    @pl.when(i == 0)
    def _init():
        std_row = jnp.sqrt(var_row_ref[...]) + 1e-8
        inv_std_row = 1.0 / std_row
        vtil_ref[...] = inv_std_row * inv_std_row
        inv_std_col = 1.0 / (jnp.sqrt(var_col_ref[...]) + 1e-8)
        a_col = a_col_ref[...]
        cn = jnp.sqrt(jnp.sum(a_col * a_col, axis=0, keepdims=True))
        bhat = a_col * (1.0 / jnp.maximum(cn, 1e-12)) * inv_std_col
        bhat_ref[...] = bhat.astype(jnp.bfloat16)
        a_row = a_row_ref[...]
        rn = jnp.sqrt(jnp.sum(a_row * a_row, axis=1, keepdims=True))
        atil = a_row * (1.0 / jnp.maximum(rn, 1e-12)) * std_row
        atil_ref[...] = atil.astype(jnp.bfloat16)

    x = x_ref[...]
    d = mean_ref[...] - x
    m30 = jnp.dot(d.astype(jnp.bfloat16), bhat_ref[...],
                  preferred_element_type=jnp.float32)
    t = (d * d) * vtil_ref[...]
    n2w = jnp.sum(t, axis=1, keepdims=True)
    rinv = jax.lax.rsqrt(jnp.maximum(n2w, 1e-24))

    r0 = jax.lax.rem(i, jnp.int32(_S // _BS)) * _BS
    s_idx = r0 + jax.lax.broadcasted_iota(jnp.int32, (_BS, 1), 0)
    rel = s_idx.astype(jnp.float32) / jnp.float32(_S)
    ptype = jnp.where(rel < 0.3, 0, jnp.where(rel < 0.7, 1, 2))
    col = jax.lax.broadcasted_iota(jnp.int32, (1, _TK), 1)
    valid = (col // _K) == ptype
    masked = jnp.where(valid, m30, jnp.inf)
    mn = jnp.min(masked, axis=1, keepdims=True)
    first = jnp.min(jnp.where(masked == mn, col, _TK), axis=1, keepdims=True)
    onehot = (col == first).astype(jnp.bfloat16)
    nst = jnp.dot(onehot, atil_ref[...],
                  preferred_element_type=jnp.float32)

    ids = ids_ref[...]
    is_marker = jax.lax.rem(ids, jnp.int32(500)) == 0
    wgt = jnp.where(is_marker, 5.0, jnp.where(rel > 0.7, 2.0, 1.0))
    alpha = _ALPHA_BASE * wgt.astype(jnp.float32)

    sims_max = -mn * rinv
    g2 = jnp.maximum(2.0 - 2.0 * sims_max, 1e-24)
    s = jnp.minimum(alpha, _MAX_DELTA * jax.lax.rsqrt(g2))
    u = s * rinv
    o_ref[...] = x + u * d + s * nst


@jax.jit
def kernel(hidden_states, input_ids, attractors, running_mean, running_var):
    Bb, Ss, Dd = hidden_states.shape
    x2 = hidden_states.reshape(Bb * Ss, Dd)
    ids2 = input_ids.reshape(Bb * Ss, 1)
    a_row = attractors.reshape(_NTYPES * _K, Dd)
    a_col = a_row.T
    mean2 = running_mean.reshape(1, Dd)
    var_row = running_var.reshape(1, Dd)
    var_col = running_var.reshape(Dd, 1)
    nblk = (Bb * Ss) // _BS
    out = pl.pallas_call(
        _fused_kernel,
        grid=(nblk,),
        in_specs=[
            pl.BlockSpec((_BS, Dd), lambda i: (i, 0)),
            pl.BlockSpec((_BS, 1), lambda i: (i, 0)),
            pl.BlockSpec((_TK, Dd), lambda i: (0, 0)),
            pl.BlockSpec((Dd, _TK), lambda i: (0, 0)),
            pl.BlockSpec((1, Dd), lambda i: (0, 0)),
            pl.BlockSpec((1, Dd), lambda i: (0, 0)),
            pl.BlockSpec((Dd, 1), lambda i: (0, 0)),
        ],
        out_specs=pl.BlockSpec((_BS, Dd), lambda i: (i, 0)),
        out_shape=jax.ShapeDtypeStruct((Bb * Ss, Dd), jnp.float32),
        scratch_shapes=[
            pltpu.VMEM((Dd, _TK), jnp.bfloat16),
            pltpu.VMEM((_TK, Dd), jnp.bfloat16),
            pltpu.VMEM((1, Dd), jnp.float32),
        ],
    )(x2, ids2, a_row, a_col, mean2, var_row, var_col)
    return out.reshape(Bb, Ss, Dd)


# speedup vs baseline: 5.9829x; 1.0241x over previous
import jax
import jax.numpy as jnp
from jax.experimental import pallas as pl
from jax.experimental.pallas import tpu as pltpu

_B, _S, _DIM, _K, _NTYPES, _VOCAB = 4, 4096, 2048, 10, 3, 32000
_ALPHA_BASE, _MAX_DELTA = 0.3, 0.5
_TK = _NTYPES * _K

_BS = 512


def _fused_kernel(x_ref, ids_ref, alpha0_ref, bias_ref, a_row_ref, a_col_ref,
                  mean_ref, var_row_ref, var_col_ref, o_ref,
                  bhat_ref, atil_ref, vtil_ref):
    i = pl.program_id(0)

    @pl.when(i == 0)
    def _init():
        std_row = jnp.sqrt(var_row_ref[...]) + 1e-8
        inv_std_col = 1.0 / (jnp.sqrt(var_col_ref[...]) + 1e-8)
        vtil_ref[...] = (inv_std_col * inv_std_col).astype(jnp.bfloat16)
        a_col = a_col_ref[...]
        cn = jnp.sqrt(jnp.sum(a_col * a_col, axis=0, keepdims=True))
        bhat = a_col * (1.0 / jnp.maximum(cn, 1e-12)) * inv_std_col
        bhat_ref[...] = bhat.astype(jnp.bfloat16)
        a_row = a_row_ref[...]
        rn = jnp.sqrt(jnp.sum(a_row * a_row, axis=1, keepdims=True))
        atil = a_row * (1.0 / jnp.maximum(rn, 1e-12)) * std_row
        atil_ref[...] = atil.astype(jnp.bfloat16)

    x = x_ref[...]
    d = mean_ref[...] - x
    db = d.astype(jnp.bfloat16)
    db2 = db * db
    m30 = jnp.dot(db, bhat_ref[...],
                  preferred_element_type=jnp.float32)
    n2w = jnp.dot(db2, vtil_ref[...],
                  preferred_element_type=jnp.float32)
    rinv = jax.lax.rsqrt(jnp.maximum(n2w, 1e-24))

    masked = m30 + bias_ref[...]
    mn = jnp.min(masked, axis=1, keepdims=True)

    ids = ids_ref[...]
    is_marker = jax.lax.rem(ids, jnp.int32(500)) == 0
    alpha = jnp.where(is_marker, _ALPHA_BASE * 5.0, alpha0_ref[...])

    sims_max = -mn * rinv
    g2 = jnp.maximum(2.0 - 2.0 * sims_max, 1e-24)
    s = jnp.minimum(alpha, _MAX_DELTA * jax.lax.rsqrt(g2))
    u = s * rinv
    onehot_s = jnp.where(masked == mn, s, 0.0).astype(jnp.bfloat16)
    nst_s = jnp.dot(onehot_s, atil_ref[...],
                    preferred_element_type=jnp.float32)
    o_ref[...] = x + u * d + nst_s


@jax.jit
def kernel(hidden_states, input_ids, attractors, running_mean, running_var):
    Bb, Ss, Dd = hidden_states.shape
    n = Bb * Ss
    x2 = hidden_states.reshape(n, Dd)
    ids2 = input_ids.reshape(n, 1)
    a_row = attractors.reshape(_NTYPES * _K, Dd)
    a_col = a_row.T
    mean2 = running_mean.reshape(1, Dd)
    var_row = running_var.reshape(1, Dd)
    var_col = running_var.reshape(Dd, 1)

    rel = (jnp.arange(Ss, dtype=jnp.float32) / Ss)
    ptype = jnp.where(rel < 0.3, 0, jnp.where(rel < 0.7, 1, 2))
    grp = jnp.arange(_TK, dtype=jnp.int32) // _K
    bias_s = jnp.where(grp[None, :] == ptype[:, None], 0.0, jnp.inf)
    bias = jnp.broadcast_to(bias_s[None], (Bb, Ss, _TK)).reshape(n, _TK)
    alpha0_s = _ALPHA_BASE * jnp.where(rel > 0.7, 2.0, 1.0)
    alpha0 = jnp.broadcast_to(alpha0_s[None], (Bb, Ss)).reshape(n, 1)

    nblk = n // _BS
    out = pl.pallas_call(
        _fused_kernel,
        grid=(nblk,),
        in_specs=[
            pl.BlockSpec((_BS, Dd), lambda i: (i, 0)),
            pl.BlockSpec((_BS, 1), lambda i: (i, 0)),
            pl.BlockSpec((_BS, 1), lambda i: (i, 0)),
            pl.BlockSpec((_BS, _TK), lambda i: (i, 0)),
            pl.BlockSpec((_TK, Dd), lambda i: (0, 0)),
            pl.BlockSpec((Dd, _TK), lambda i: (0, 0)),
            pl.BlockSpec((1, Dd), lambda i: (0, 0)),
            pl.BlockSpec((1, Dd), lambda i: (0, 0)),
            pl.BlockSpec((Dd, 1), lambda i: (0, 0)),
        ],
        out_specs=pl.BlockSpec((_BS, Dd), lambda i: (i, 0)),
        out_shape=jax.ShapeDtypeStruct((n, Dd), jnp.float32),
        scratch_shapes=[
            pltpu.VMEM((Dd, _TK), jnp.bfloat16),
            pltpu.VMEM((_TK, Dd), jnp.bfloat16),
            pltpu.VMEM((Dd, 1), jnp.bfloat16),
        ],
    )(x2, ids2, alpha0, bias, a_row, a_col, mean2, var_row, var_col)
    return out.reshape(Bb, Ss, Dd)


# speedup vs baseline: 6.1866x; 1.0340x over previous
import jax
import jax.numpy as jnp
from jax.experimental import pallas as pl
from jax.experimental.pallas import tpu as pltpu

_B, _S, _DIM, _K, _NTYPES, _VOCAB = 4, 4096, 2048, 10, 3, 32000
_ALPHA_BASE, _MAX_DELTA = 0.3, 0.5
_TK = _NTYPES * _K

_BS = 1024


def _fused_kernel(x_ref, ids_ref, alpha0_ref, bias_ref, a_row_ref, a_col_ref,
                  mean_ref, var_row_ref, var_col_ref, o_ref,
                  bhat_ref, atil_ref, vtil_ref):
    i = pl.program_id(0)

    @pl.when(i == 0)
    def _init():
        std_row = jnp.sqrt(var_row_ref[...]) + 1e-8
        inv_std_col = 1.0 / (jnp.sqrt(var_col_ref[...]) + 1e-8)
        vtil_ref[...] = (inv_std_col * inv_std_col).astype(jnp.bfloat16)
        a_col = a_col_ref[...]
        cn = jnp.sqrt(jnp.sum(a_col * a_col, axis=0, keepdims=True))
        bhat = a_col * (1.0 / jnp.maximum(cn, 1e-12)) * inv_std_col
        bhat_ref[...] = bhat.astype(jnp.bfloat16)
        a_row = a_row_ref[...]
        rn = jnp.sqrt(jnp.sum(a_row * a_row, axis=1, keepdims=True))
        atil = a_row * (1.0 / jnp.maximum(rn, 1e-12)) * std_row
        atil_ref[...] = atil.astype(jnp.bfloat16)

    x = x_ref[...]
    db = (mean_ref[...] - x).astype(jnp.bfloat16)
    db2 = db * db
    m30 = jnp.dot(db, bhat_ref[...],
                  preferred_element_type=jnp.float32)
    n2w = jnp.dot(db2, vtil_ref[...],
                  preferred_element_type=jnp.float32)
    rinv = jax.lax.rsqrt(jnp.maximum(n2w, 1e-24))

    masked = m30 + bias_ref[...]
    mn = jnp.min(masked, axis=1, keepdims=True)

    ids = ids_ref[...]
    is_marker = jax.lax.rem(ids, jnp.int32(500)) == 0
    alpha = jnp.where(is_marker, _ALPHA_BASE * 5.0, alpha0_ref[...])

    sims_max = -mn * rinv
    g2 = jnp.maximum(2.0 - 2.0 * sims_max, 1e-24)
    s = jnp.minimum(alpha, _MAX_DELTA * jax.lax.rsqrt(g2))
    u = s * rinv
    onehot_s = jnp.where(masked == mn, s, 0.0).astype(jnp.bfloat16)
    nst_s = jnp.dot(onehot_s, atil_ref[...],
                    preferred_element_type=jnp.float32)
    o_ref[...] = x * (1.0 - u) + u * mean_ref[...] + nst_s


@jax.jit
def kernel(hidden_states, input_ids, attractors, running_mean, running_var):
    Bb, Ss, Dd = hidden_states.shape
    n = Bb * Ss
    x2 = hidden_states.reshape(n, Dd)
    ids2 = input_ids.reshape(n, 1)
    a_row = attractors.reshape(_NTYPES * _K, Dd)
    a_col = a_row.T
    mean2 = running_mean.reshape(1, Dd)
    var_row = running_var.reshape(1, Dd)
    var_col = running_var.reshape(Dd, 1)

    rel = (jnp.arange(Ss, dtype=jnp.float32) / Ss)
    ptype = jnp.where(rel < 0.3, 0, jnp.where(rel < 0.7, 1, 2))
    grp = jnp.arange(_TK, dtype=jnp.int32) // _K
    bias_s = jnp.where(grp[None, :] == ptype[:, None], 0.0, jnp.inf)
    bias = jnp.broadcast_to(bias_s[None], (Bb, Ss, _TK)).reshape(n, _TK)
    alpha0_s = _ALPHA_BASE * jnp.where(rel > 0.7, 2.0, 1.0)
    alpha0 = jnp.broadcast_to(alpha0_s[None], (Bb, Ss)).reshape(n, 1)

    nblk = n // _BS
    out = pl.pallas_call(
        _fused_kernel,
        grid=(nblk,),
        in_specs=[
            pl.BlockSpec((_BS, Dd), lambda i: (i, 0)),
            pl.BlockSpec((_BS, 1), lambda i: (i, 0)),
            pl.BlockSpec((_BS, 1), lambda i: (i, 0)),
            pl.BlockSpec((_BS, _TK), lambda i: (i, 0)),
            pl.BlockSpec((_TK, Dd), lambda i: (0, 0)),
            pl.BlockSpec((Dd, _TK), lambda i: (0, 0)),
            pl.BlockSpec((1, Dd), lambda i: (0, 0)),
            pl.BlockSpec((1, Dd), lambda i: (0, 0)),
            pl.BlockSpec((Dd, 1), lambda i: (0, 0)),
        ],
        out_specs=pl.BlockSpec((_BS, Dd), lambda i: (i, 0)),
        out_shape=jax.ShapeDtypeStruct((n, Dd), jnp.float32),
        scratch_shapes=[
            pltpu.VMEM((Dd, _TK), jnp.bfloat16),
            pltpu.VMEM((_TK, Dd), jnp.bfloat16),
            pltpu.VMEM((Dd, 1), jnp.bfloat16),
        ],
    )(x2, ids2, alpha0, bias, a_row, a_col, mean2, var_row, var_col)
    return out.reshape(Bb, Ss, Dd)


# speedup vs baseline: 6.9766x; 1.1277x over previous
import jax
import jax.numpy as jnp
from jax import lax
from jax.experimental import pallas as pl
from jax.experimental.pallas import tpu as pltpu

_B, _S, _DIM, _K, _NTYPES, _VOCAB = 4, 4096, 2048, 10, 3, 32000
_ALPHA_BASE, _MAX_DELTA = 0.3, 0.5
_TK = _NTYPES * _K
_N = _B * _S

_BS = 1024

_CONTRACT_RHS1 = (((1,), (1,)), ((), ()))


def _fused_kernel(x_ref, ids_ref, a_row_ref, mean_ref, var_row_ref,
                  var_col_ref, o_ref,
                  bhat_ref, atil_ref, vtil_ref, bias_ref, alpha0_ref):
    i = pl.program_id(0)

    @pl.when(i == 0)
    def _init():
        std_row = jnp.sqrt(var_row_ref[...]) + 1e-8
        inv_std_row = 1.0 / std_row
        inv_std_col = 1.0 / (jnp.sqrt(var_col_ref[...]) + 1e-8)
        vtil_ref[...] = (inv_std_col * inv_std_col).astype(jnp.bfloat16)
        a_row = a_row_ref[...]
        rn = jnp.sqrt(jnp.sum(a_row * a_row, axis=1, keepdims=True))
        a_n = a_row * (1.0 / jnp.maximum(rn, 1e-12))
        bhat_ref[...] = (a_n * inv_std_row).astype(jnp.bfloat16)
        atil_ref[...] = (a_n * std_row).astype(jnp.bfloat16)
        s_all = lax.rem(lax.broadcasted_iota(jnp.int32, (_N, _TK), 0),
                        jnp.int32(_S))
        grp = lax.broadcasted_iota(jnp.int32, (_N, _TK), 1) // _K
        ptype = jnp.where(s_all <= 1228, 0, jnp.where(s_all <= 2867, 1, 2))
        bias_ref[...] = jnp.where(grp == ptype, 0.0, jnp.inf)
        s_col = lax.rem(lax.broadcasted_iota(jnp.int32, (_N, 1), 0),
                        jnp.int32(_S))
        alpha0_ref[...] = jnp.where(s_col >= 2868,
                                    _ALPHA_BASE * 2.0, _ALPHA_BASE)

    x = x_ref[...]
    db = (mean_ref[...] - x).astype(jnp.bfloat16)
    db2 = db * db
    m30 = lax.dot_general(db, bhat_ref[...], _CONTRACT_RHS1,
                          preferred_element_type=jnp.float32)
    n2w = jnp.dot(db2, vtil_ref[...],
                  preferred_element_type=jnp.float32)
    rinv = lax.rsqrt(jnp.maximum(n2w, 1e-24))

    masked = m30 + bias_ref[pl.ds(i * _BS, _BS), :]
    mn = jnp.min(masked, axis=1, keepdims=True)

    ids = ids_ref[...]
    is_marker = lax.rem(ids, jnp.int32(500)) == 0
    alpha = jnp.where(is_marker, _ALPHA_BASE * 5.0,
                      alpha0_ref[pl.ds(i * _BS, _BS), :])

    sims_max = -mn * rinv
    g2 = jnp.maximum(2.0 - 2.0 * sims_max, 1e-24)
    s = jnp.minimum(alpha, _MAX_DELTA * lax.rsqrt(g2))
    u = s * rinv
    onehot_s = jnp.where(masked == mn, s, 0.0).astype(jnp.bfloat16)
    nst_s = jnp.dot(onehot_s, atil_ref[...],
                    preferred_element_type=jnp.float32)
    o_ref[...] = x * (1.0 - u) + u * mean_ref[...] + nst_s


@jax.jit
def kernel(hidden_states, input_ids, attractors, running_mean, running_var):
    Bb, Ss, Dd = hidden_states.shape
    n = Bb * Ss
    x2 = hidden_states.reshape(n, Dd)
    ids2 = input_ids.reshape(n, 1)
    a_row = attractors.reshape(_NTYPES * _K, Dd)
    mean2 = running_mean.reshape(1, Dd)
    var_row = running_var.reshape(1, Dd)
    var_col = running_var.reshape(Dd, 1)
    out = pl.pallas_call(
        _fused_kernel,
        grid=(n // _BS,),
        in_specs=[
            pl.BlockSpec((_BS, Dd), lambda i: (i, 0)),
            pl.BlockSpec((_BS, 1), lambda i: (i, 0)),
            pl.BlockSpec((_TK, Dd), lambda i: (0, 0)),
            pl.BlockSpec((1, Dd), lambda i: (0, 0)),
            pl.BlockSpec((1, Dd), lambda i: (0, 0)),
            pl.BlockSpec((Dd, 1), lambda i: (0, 0)),
        ],
        out_specs=pl.BlockSpec((_BS, Dd), lambda i: (i, 0)),
        out_shape=jax.ShapeDtypeStruct((n, Dd), jnp.float32),
        scratch_shapes=[
            pltpu.VMEM((_TK, Dd), jnp.bfloat16),
            pltpu.VMEM((_TK, Dd), jnp.bfloat16),
            pltpu.VMEM((Dd, 1), jnp.bfloat16),
            pltpu.VMEM((_N, _TK), jnp.float32),
            pltpu.VMEM((_N, 1), jnp.float32),
        ],
    )(x2, ids2, a_row, mean2, var_row, var_col)
    return out.reshape(Bb, Ss, Dd)
